# baseline (speedup 1.0000x reference)
import jax
import jax.numpy as jnp
from jax import lax
from jax.experimental import pallas as pl
from jax.experimental.pallas import tpu as pltpu

_f32 = jnp.float32
_i32 = jnp.int32


def _pcall(body, **kw):
    return pl.pallas_call(body, **kw)


def _fps(xyz, npoint):
    B, N, _ = xyz.shape
    xs = xyz[:, :, 0]
    ys = xyz[:, :, 1]
    zs = xyz[:, :, 2]

    def body(x_ref, y_ref, z_ref, out_ref):
        X = x_ref[...]
        Y = y_ref[...]
        Z = z_ref[...]
        iota = lax.broadcasted_iota(_i32, (B, N), 1)
        iop = lax.broadcasted_iota(_i32, (B, npoint), 1)

        def step(i, state):
            dists, far, idxs = state
            idxs = idxs + ((iop == i).astype(_i32)
                           * jnp.broadcast_to(far, (B, npoint)))
            sel = iota == jnp.broadcast_to(far, (B, N))
            cx = jnp.sum(jnp.where(sel, X, 0.0), axis=1, keepdims=True)
            cy = jnp.sum(jnp.where(sel, Y, 0.0), axis=1, keepdims=True)
            cz = jnp.sum(jnp.where(sel, Z, 0.0), axis=1, keepdims=True)
            dx = X - cx
            dy = Y - cy
            dz = Z - cz
            d = dx * dx + dy * dy + dz * dz
            dists = jnp.minimum(dists, d)
            m = jnp.max(dists, axis=1, keepdims=True)
            far = jnp.min(jnp.where(dists == jnp.broadcast_to(m, (B, N)),
                                    iota, N), axis=1,
                          keepdims=True).astype(_i32)
            return dists, far, idxs

        dists0 = X * 0.0 + 1e10
        far0 = (jnp.max(X * 0.0, axis=1, keepdims=True)).astype(_i32)
        idxs0 = (X[:, :npoint] * 0.0).astype(_i32)
        _, _, idxs = lax.fori_loop(0, npoint, step, (dists0, far0, idxs0))
        out_ref[...] = idxs

    return _pcall(
        body,
        out_shape=jax.ShapeDtypeStruct((B, npoint), _i32),
    )(xs, ys, zs)


def _knn(xyzT, fps_col, S, K):
    B, _, N = xyzT.shape

    def body(xt_ref, fi_ref, knn_ref, nx_ref):
        A = xt_ref[0]
        X = A[0:1, :]
        Y = A[1:2, :]
        Z = A[2:3, :]
        idx = fi_ref[0]
        iotaSN = lax.broadcasted_iota(_i32, (S, N), 1)
        sel = iotaSN == jnp.broadcast_to(idx, (S, N))
        Xb = jnp.broadcast_to(X, (S, N))
        Yb = jnp.broadcast_to(Y, (S, N))
        Zb = jnp.broadcast_to(Z, (S, N))
        cx = jnp.sum(jnp.where(sel, Xb, 0.0), axis=1, keepdims=True)
        cy = jnp.sum(jnp.where(sel, Yb, 0.0), axis=1, keepdims=True)
        cz = jnp.sum(jnp.where(sel, Zb, 0.0), axis=1, keepdims=True)
        C = jnp.concatenate([cx, cy, cz], axis=1)
        dot = lax.dot_general(C, A, (((1,), (0,)), ((), ())),
                              preferred_element_type=_f32)
        sqC = jnp.sum(C * C, axis=1, keepdims=True)
        X2 = A * A
        sqX = (X2[0:1] + X2[1:2]) + X2[2:3]
        D = (-2.0 * dot + sqC) + sqX
        iotaK = lax.broadcasted_iota(_i32, (S, K), 1)

        def step(k, st):
            D, knn = st
            m = jnp.min(D, axis=1, keepdims=True)
            j = jnp.min(jnp.where(D == jnp.broadcast_to(m, (S, N)),
                                  iotaSN, N), axis=1,
                        keepdims=True).astype(_i32)
            knn = knn + ((iotaK == k).astype(_i32)
                         * jnp.broadcast_to(j, (S, K)))
            D = jnp.where(iotaSN == jnp.broadcast_to(j, (S, N)), jnp.inf, D)
            return D, knn

        knn0 = (D[:, :K] * 0.0).astype(_i32)
        _, knn = lax.fori_loop(0, K, step, (D, knn0))
        knn_ref[...] = knn[None]
        nx_ref[...] = C[None]

    return _pcall(
        body,
        grid=(B,),
        in_specs=[
            pl.BlockSpec((1, 3, N), lambda b: (b, 0, 0)),
            pl.BlockSpec((1, S, 1), lambda b: (b, 0, 0)),
        ],
        out_specs=[
            pl.BlockSpec((1, S, K), lambda b: (b, 0, 0)),
            pl.BlockSpec((1, S, 3), lambda b: (b, 0, 0)),
        ],
        out_shape=[
            jax.ShapeDtypeStruct((B, S, K), _i32),
            jax.ShapeDtypeStruct((B, S, 3), _f32),
        ],
        compiler_params=pltpu.CompilerParams(
            dimension_semantics=("parallel",)),
    )(xyzT, fps_col)


def _group_mm(pts, knn_flat, cent_flat, W, b, CH):
    B, N, C = pts.shape
    R = knn_flat.shape[1]
    Cout = W.shape[1]
    NC = R // CH

    def body(p_ref, k_ref, c_ref, w_ref, b_ref, y_ref, s_ref, ss_ref):
        c = pl.program_id(1)
        P = p_ref[0]
        idx = k_ref[0]
        cent = c_ref[0]
        onehot = (lax.broadcasted_iota(_i32, (CH, N), 1)
                  == jnp.broadcast_to(idx, (CH, N))).astype(_f32)
        G = jnp.dot(onehot, P, preferred_element_type=_f32,
                    precision=lax.Precision.HIGHEST)
        grouped = jnp.concatenate([G[:, :3] - cent, G[:, 3:]], axis=1)
        y = jnp.dot(grouped, w_ref[...], preferred_element_type=_f32) + b_ref[...]
        y_ref[...] = y[None]
        sv = jnp.sum(y, axis=0, keepdims=True)[None]
        sq = jnp.sum(y * y, axis=0, keepdims=True)[None]

        @pl.when(c == 0)
        def _():
            s_ref[...] = sv
            ss_ref[...] = sq

        @pl.when(c != 0)
        def _():
            s_ref[...] = s_ref[...] + sv
            ss_ref[...] = ss_ref[...] + sq

    return _pcall(
        body,
        grid=(B, NC),
        in_specs=[
            pl.BlockSpec((1, N, C), lambda bb, cc: (bb, 0, 0)),
            pl.BlockSpec((1, CH, 1), lambda bb, cc: (bb, cc, 0)),
            pl.BlockSpec((1, CH, 3), lambda bb, cc: (bb, cc, 0)),
            pl.BlockSpec((C, Cout), lambda bb, cc: (0, 0)),
            pl.BlockSpec((1, Cout), lambda bb, cc: (0, 0)),
        ],
        out_specs=[
            pl.BlockSpec((1, CH, Cout), lambda bb, cc: (bb, cc, 0)),
            pl.BlockSpec((1, 1, Cout), lambda bb, cc: (bb, 0, 0)),
            pl.BlockSpec((1, 1, Cout), lambda bb, cc: (bb, 0, 0)),
        ],
        out_shape=[
            jax.ShapeDtypeStruct((B, R, Cout), _f32),
            jax.ShapeDtypeStruct((B, 1, Cout), _f32),
            jax.ShapeDtypeStruct((B, 1, Cout), _f32),
        ],
        compiler_params=pltpu.CompilerParams(
            dimension_semantics=("parallel", "arbitrary")),
    )(pts, knn_flat, cent_flat, W, b)


def _bn_mm(y, mean, rstd, gamma, beta, W, b, CH):
    B, R, Cin = y.shape
    Cout = W.shape[1]
    NC = R // CH

    def body(y_ref, m_ref, r_ref, g_ref, e_ref, w_ref, b_ref,
             o_ref, s_ref, ss_ref):
        c = pl.program_id(1)
        x = y_ref[0]
        h = (x - m_ref[...]) * r_ref[...] * g_ref[...] + e_ref[...]
        h = jnp.maximum(h, 0.0)
        y2 = jnp.dot(h, w_ref[...], preferred_element_type=_f32) + b_ref[...]
        o_ref[...] = y2[None]
        sv = jnp.sum(y2, axis=0, keepdims=True)[None]
        sq = jnp.sum(y2 * y2, axis=0, keepdims=True)[None]

        @pl.when(c == 0)
        def _():
            s_ref[...] = sv
            ss_ref[...] = sq

        @pl.when(c != 0)
        def _():
            s_ref[...] = s_ref[...] + sv
            ss_ref[...] = ss_ref[...] + sq

    return _pcall(
        body,
        grid=(B, NC),
        in_specs=[
            pl.BlockSpec((1, CH, Cin), lambda bb, cc: (bb, cc, 0)),
            pl.BlockSpec((1, Cin), lambda bb, cc: (0, 0)),
            pl.BlockSpec((1, Cin), lambda bb, cc: (0, 0)),
            pl.BlockSpec((1, Cin), lambda bb, cc: (0, 0)),
            pl.BlockSpec((1, Cin), lambda bb, cc: (0, 0)),
            pl.BlockSpec((Cin, Cout), lambda bb, cc: (0, 0)),
            pl.BlockSpec((1, Cout), lambda bb, cc: (0, 0)),
        ],
        out_specs=[
            pl.BlockSpec((1, CH, Cout), lambda bb, cc: (bb, cc, 0)),
            pl.BlockSpec((1, 1, Cout), lambda bb, cc: (bb, 0, 0)),
            pl.BlockSpec((1, 1, Cout), lambda bb, cc: (bb, 0, 0)),
        ],
        out_shape=[
            jax.ShapeDtypeStruct((B, R, Cout), _f32),
            jax.ShapeDtypeStruct((B, 1, Cout), _f32),
            jax.ShapeDtypeStruct((B, 1, Cout), _f32),
        ],
        compiler_params=pltpu.CompilerParams(
            dimension_semantics=("parallel", "arbitrary")),
    )(y, mean, rstd, gamma, beta, W, b)


def _bn_pool(y, mean, rstd, gamma, beta, S, K):
    B, R, C = y.shape

    def body(y_ref, m_ref, r_ref, g_ref, e_ref, o_ref):
        x = y_ref[0]
        h = (x - m_ref[...]) * r_ref[...] * g_ref[...] + e_ref[...]
        h = jnp.maximum(h, 0.0)
        o_ref[...] = jnp.max(h.reshape(S, K, C), axis=1)[None]

    return _pcall(
        body,
        grid=(B,),
        in_specs=[
            pl.BlockSpec((1, R, C), lambda bb: (bb, 0, 0)),
            pl.BlockSpec((1, C), lambda bb: (0, 0)),
            pl.BlockSpec((1, C), lambda bb: (0, 0)),
            pl.BlockSpec((1, C), lambda bb: (0, 0)),
            pl.BlockSpec((1, C), lambda bb: (0, 0)),
        ],
        out_specs=pl.BlockSpec((1, S, C), lambda bb: (bb, 0, 0)),
        out_shape=jax.ShapeDtypeStruct((B, S, C), _f32),
        compiler_params=pltpu.CompilerParams(
            dimension_semantics=("parallel",)),
    )(y, mean, rstd, gamma, beta)


def _tail(nx2, f2, sa3, head):
    B, S, _ = nx2.shape
    (w1, b1, g1, e1), (w2, b2, g2, e2), (w3, b3, g3, e3) = sa3
    (h1w, h1b, h1g, h1e), (h2w, h2b, h2g, h2e), (h3w, h3b, _, _) = head

    def bn_all(ymat):
        mean = jnp.mean(ymat, axis=0, keepdims=True)
        var = jnp.mean((ymat - mean) * (ymat - mean), axis=0, keepdims=True)
        return mean, lax.rsqrt(var + 1e-5)

    def body(nx_ref, f_ref,
             w1_ref, b1_ref, g1_ref, e1_ref,
             w2_ref, b2_ref, g2_ref, e2_ref,
             w3_ref, b3_ref, g3_ref, e3_ref,
             h1w_ref, h1b_ref, h1g_ref, h1e_ref,
             h2w_ref, h2b_ref, h2g_ref, h2e_ref,
             h3w_ref, h3b_ref, o_ref):
        g = jnp.concatenate([nx_ref[...], f_ref[...]], axis=2)
        x = g.reshape(B * S, g.shape[2])

        for w_r, b_r, g_r, e_r in (
                (w1_ref, b1_ref, g1_ref, e1_ref),
                (w2_ref, b2_ref, g2_ref, e2_ref),
                (w3_ref, b3_ref, g3_ref, e3_ref)):
            x = jnp.dot(x, w_r[...], preferred_element_type=_f32) + b_r[...]
            mean, rstd = bn_all(x)
            x = jnp.maximum((x - mean) * rstd * g_r[...] + e_r[...], 0.0)

        x = jnp.max(x.reshape(B, S, x.shape[1]), axis=1)

        for w_r, b_r, g_r, e_r in (
                (h1w_ref, h1b_ref, h1g_ref, h1e_ref),
                (h2w_ref, h2b_ref, h2g_ref, h2e_ref)):
            x = jnp.dot(x, w_r[...], preferred_element_type=_f32) + b_r[...]
            mean, rstd = bn_all(x)
            x = (x - mean) * rstd * g_r[...] + e_r[...]

        x = jnp.dot(x, h3w_ref[...], preferred_element_type=_f32) + h3b_ref[...]
        x = x - jnp.max(x, axis=1, keepdims=True)
        x = x - jnp.log(jnp.sum(jnp.exp(x), axis=1, keepdims=True))
        o_ref[...] = x

    args = (nx2, f2,
            w1, b1.reshape(1, -1), g1.reshape(1, -1), e1.reshape(1, -1),
            w2, b2.reshape(1, -1), g2.reshape(1, -1), e2.reshape(1, -1),
            w3, b3.reshape(1, -1), g3.reshape(1, -1), e3.reshape(1, -1),
            h1w, h1b.reshape(1, -1), h1g.reshape(1, -1), h1e.reshape(1, -1),
            h2w, h2b.reshape(1, -1), h2g.reshape(1, -1), h2e.reshape(1, -1),
            h3w, h3b.reshape(1, -1))
    return _pcall(
        body,
        out_shape=jax.ShapeDtypeStruct((B, h3w.shape[1]), _f32),
    )(*args)


def _stats(s, ss, n):
    tot = jnp.sum(s, axis=0)
    tot2 = jnp.sum(ss, axis=0)
    mean = tot / n
    var = tot2 / n - mean * mean
    return mean, lax.rsqrt(var + 1e-5)


def _sa_stage(pts_xyz, pts_feats, layers, npoint, K, CH):
    B, N, _ = pts_xyz.shape
    fps_idx = _fps(pts_xyz, npoint)
    knn, new_xyz = _knn(pts_xyz.transpose(0, 2, 1),
                        fps_idx.reshape(B, npoint, 1), npoint, K)
    pts = jnp.concatenate([pts_xyz, pts_feats], axis=2)
    R = npoint * K
    cent = jnp.broadcast_to(new_xyz[:, :, None, :],
                            (B, npoint, K, 3)).reshape(B, R, 3)
    knn_flat = knn.reshape(B, R, 1)

    (w1, b1, g1, e1) = layers[0]
    y, s, ss = _group_mm(pts, knn_flat, cent, w1, b1.reshape(1, -1), CH)
    n = B * R
    for (w, b, g, e) in layers[1:]:
        mean, rstd = _stats(s, ss, n)
        prev_g, prev_e = g1, e1
        y, s, ss = _bn_mm(y, mean, rstd, prev_g.reshape(1, -1),
                          prev_e.reshape(1, -1), w, b.reshape(1, -1), CH)
        g1, e1 = g, e
    mean, rstd = _stats(s, ss, n)
    pooled = _bn_pool(y, mean, rstd, g1.reshape(1, -1), e1.reshape(1, -1),
                      npoint, K)
    return new_xyz, pooled


def kernel(xyz, normals, params):
    sa = params['sa']
    head = params['head']
    nx1, f1 = _sa_stage(xyz, normals, sa[0], npoint=512, K=32, CH=2048)
    return jnp.sum(nx1) + jnp.sum(f1)


# speedup vs baseline: 1.3038x; 1.3038x over previous
import functools

import jax
import jax.numpy as jnp
from jax import lax
from jax.experimental import pallas as pl
from jax.experimental.pallas import tpu as pltpu
from jax.experimental.pallas import tpu_sc as plsc

_f32 = jnp.float32
_i32 = jnp.int32


def _pcall(body, **kw):
    return pl.pallas_call(body, **kw)


def _fps(xyz, npoint):
    B, N, _ = xyz.shape
    xs = xyz[:, :, 0]
    ys = xyz[:, :, 1]
    zs = xyz[:, :, 2]

    def body(x_ref, y_ref, z_ref, out_ref):
        X = x_ref[...]
        Y = y_ref[...]
        Z = z_ref[...]
        iota = lax.broadcasted_iota(_i32, (B, N), 1)
        iop = lax.broadcasted_iota(_i32, (B, npoint), 1)

        def step(i, state):
            dists, far, idxs = state
            idxs = idxs + ((iop == i).astype(_i32)
                           * jnp.broadcast_to(far, (B, npoint)))
            sel = iota == jnp.broadcast_to(far, (B, N))
            cx = jnp.sum(jnp.where(sel, X, 0.0), axis=1, keepdims=True)
            cy = jnp.sum(jnp.where(sel, Y, 0.0), axis=1, keepdims=True)
            cz = jnp.sum(jnp.where(sel, Z, 0.0), axis=1, keepdims=True)
            dx = X - cx
            dy = Y - cy
            dz = Z - cz
            d = dx * dx + dy * dy + dz * dz
            dists = jnp.minimum(dists, d)
            m = jnp.max(dists, axis=1, keepdims=True)
            far = jnp.min(jnp.where(dists == jnp.broadcast_to(m, (B, N)),
                                    iota, N), axis=1,
                          keepdims=True).astype(_i32)
            return dists, far, idxs

        dists0 = X * 0.0 + 1e10
        far0 = (jnp.max(X * 0.0, axis=1, keepdims=True)).astype(_i32)
        idxs0 = (X[:, :npoint] * 0.0).astype(_i32)
        _, _, idxs = lax.fori_loop(0, npoint, step, (dists0, far0, idxs0))
        out_ref[...] = idxs

    return _pcall(
        body,
        out_shape=jax.ShapeDtypeStruct((B, npoint), _i32),
    )(xs, ys, zs)


def _knn(xyzT, fps_col, S, K):
    B, _, N = xyzT.shape

    def body(xt_ref, fi_ref, knn_ref, nx_ref):
        A = xt_ref[0]
        X = A[0:1, :]
        Y = A[1:2, :]
        Z = A[2:3, :]
        idx = fi_ref[0]
        iotaSN = lax.broadcasted_iota(_i32, (S, N), 1)
        sel = iotaSN == jnp.broadcast_to(idx, (S, N))
        Xb = jnp.broadcast_to(X, (S, N))
        Yb = jnp.broadcast_to(Y, (S, N))
        Zb = jnp.broadcast_to(Z, (S, N))
        cx = jnp.sum(jnp.where(sel, Xb, 0.0), axis=1, keepdims=True)
        cy = jnp.sum(jnp.where(sel, Yb, 0.0), axis=1, keepdims=True)
        cz = jnp.sum(jnp.where(sel, Zb, 0.0), axis=1, keepdims=True)
        C = jnp.concatenate([cx, cy, cz], axis=1)
        dot = lax.dot_general(C, A, (((1,), (0,)), ((), ())),
                              preferred_element_type=_f32)
        sqC = jnp.sum(C * C, axis=1, keepdims=True)
        X2 = A * A
        sqX = (X2[0:1] + X2[1:2]) + X2[2:3]
        D = (-2.0 * dot + sqC) + sqX
        iotaK = lax.broadcasted_iota(_i32, (S, K), 1)

        def step(k, st):
            D, knn = st
            m = jnp.min(D, axis=1, keepdims=True)
            j = jnp.min(jnp.where(D == jnp.broadcast_to(m, (S, N)),
                                  iotaSN, N), axis=1,
                        keepdims=True).astype(_i32)
            knn = knn + ((iotaK == k).astype(_i32)
                         * jnp.broadcast_to(j, (S, K)))
            D = jnp.where(iotaSN == jnp.broadcast_to(j, (S, N)), jnp.inf, D)
            return D, knn

        knn0 = (D[:, :K] * 0.0).astype(_i32)
        _, knn = lax.fori_loop(0, K, step, (D, knn0))
        knn_ref[...] = knn[None]
        nx_ref[...] = C[None]

    return _pcall(
        body,
        grid=(B,),
        in_specs=[
            pl.BlockSpec((1, 3, N), lambda b: (b, 0, 0)),
            pl.BlockSpec((1, S, 1), lambda b: (b, 0, 0)),
        ],
        out_specs=[
            pl.BlockSpec((1, S, K), lambda b: (b, 0, 0)),
            pl.BlockSpec((1, S, 3), lambda b: (b, 0, 0)),
        ],
        out_shape=[
            jax.ShapeDtypeStruct((B, S, K), _i32),
            jax.ShapeDtypeStruct((B, S, 3), _f32),
        ],
        compiler_params=pltpu.CompilerParams(
            dimension_semantics=("parallel",)),
    )(xyzT, fps_col)


def _sc_gather(table, idx, C):
    M = idx.shape[0]
    info = plsc.get_sparse_core_info()
    NW = info.num_cores * info.num_subcores
    per_w = M // NW
    SUB = 128
    n_sub = per_w // SUB
    mesh = plsc.VectorSubcoreMesh(core_axis_name="c", subcore_axis_name="s")

    @functools.partial(
        pl.kernel, mesh=mesh,
        out_type=jax.ShapeDtypeStruct((M, C), _f32),
        scratch_types=[
            pltpu.VMEM((SUB,), _i32),
            pltpu.VMEM((SUB, C), _f32),
            pltpu.SemaphoreType.DMA,
        ],
    )
    def k(table_hbm, idx_hbm, out_hbm, idx_v, rows_v, sem):
        wid = lax.axis_index("s") * info.num_cores + lax.axis_index("c")
        base = wid * per_w

        def body(i, carry):
            off = base + i * SUB
            pltpu.sync_copy(idx_hbm.at[pl.ds(off, SUB)], idx_v)
            pltpu.async_copy(table_hbm.at[idx_v], rows_v, sem).wait()
            pltpu.sync_copy(rows_v, out_hbm.at[pl.ds(off, SUB)])
            return carry

        lax.fori_loop(0, n_sub, body, 0)

    return k(table, idx)


def _feat_mm(x, W):
    B, N, F = x.shape
    C = W.shape[1]

    def body(x_ref, w_ref, o_ref):
        o_ref[...] = jnp.dot(x_ref[0], w_ref[...],
                             preferred_element_type=_f32)[None]

    return _pcall(
        body,
        grid=(B,),
        in_specs=[pl.BlockSpec((1, N, F), lambda b: (b, 0, 0)),
                  pl.BlockSpec((F, C), lambda b: (0, 0))],
        out_specs=pl.BlockSpec((1, N, C), lambda b: (b, 0, 0)),
        out_shape=jax.ShapeDtypeStruct((B, N, C), _f32),
        compiler_params=pltpu.CompilerParams(
            dimension_semantics=("parallel",)),
    )(x, W)


def _rel_mm(gf, gx, cent_flat, Cout, W3, b, CH):
    B, R, PAD = gf.shape
    NC = R // CH

    def body(gf_ref, gx_ref, c_ref, w_ref, b_ref, y_ref, s_ref, ss_ref):
        c = pl.program_id(1)
        rel = gx_ref[0][:, :3] - c_ref[0]
        y = (jnp.dot(rel, w_ref[...], preferred_element_type=_f32)
             + gf_ref[0][:, :Cout]) + b_ref[...]
        y_ref[...] = y[None]
        sv = jnp.sum(y, axis=0, keepdims=True)[None]
        sq = jnp.sum(y * y, axis=0, keepdims=True)[None]

        @pl.when(c == 0)
        def _():
            s_ref[...] = sv
            ss_ref[...] = sq

        @pl.when(c != 0)
        def _():
            s_ref[...] = s_ref[...] + sv
            ss_ref[...] = ss_ref[...] + sq

    return _pcall(
        body,
        grid=(B, NC),
        in_specs=[
            pl.BlockSpec((1, CH, PAD), lambda bb, cc: (bb, cc, 0)),
            pl.BlockSpec((1, CH, 128), lambda bb, cc: (bb, cc, 0)),
            pl.BlockSpec((1, CH, 3), lambda bb, cc: (bb, cc, 0)),
            pl.BlockSpec((3, Cout), lambda bb, cc: (0, 0)),
            pl.BlockSpec((1, Cout), lambda bb, cc: (0, 0)),
        ],
        out_specs=[
            pl.BlockSpec((1, CH, Cout), lambda bb, cc: (bb, cc, 0)),
            pl.BlockSpec((1, 1, Cout), lambda bb, cc: (bb, 0, 0)),
            pl.BlockSpec((1, 1, Cout), lambda bb, cc: (bb, 0, 0)),
        ],
        out_shape=[
            jax.ShapeDtypeStruct((B, R, Cout), _f32),
            jax.ShapeDtypeStruct((B, 1, Cout), _f32),
            jax.ShapeDtypeStruct((B, 1, Cout), _f32),
        ],
        compiler_params=pltpu.CompilerParams(
            dimension_semantics=("parallel", "arbitrary")),
    )(gf, gx, cent_flat, W3, b)


def _group_mm(pts, knn_flat, cent_flat, W, b, CH):
    B, N, C = pts.shape
    R = knn_flat.shape[1]
    Cout = W.shape[1]
    NC = R // CH

    def body(p_ref, k_ref, c_ref, w_ref, b_ref, y_ref, s_ref, ss_ref):
        c = pl.program_id(1)
        P = p_ref[0]
        idx = k_ref[0]
        cent = c_ref[0]
        onehot = (lax.broadcasted_iota(_i32, (CH, N), 1)
                  == jnp.broadcast_to(idx, (CH, N))).astype(_f32)
        G = jnp.dot(onehot, P, preferred_element_type=_f32,
                    precision=lax.Precision.HIGHEST)
        grouped = jnp.concatenate([G[:, :3] - cent, G[:, 3:]], axis=1)
        y = jnp.dot(grouped, w_ref[...], preferred_element_type=_f32) + b_ref[...]
        y_ref[...] = y[None]
        sv = jnp.sum(y, axis=0, keepdims=True)[None]
        sq = jnp.sum(y * y, axis=0, keepdims=True)[None]

        @pl.when(c == 0)
        def _():
            s_ref[...] = sv
            ss_ref[...] = sq

        @pl.when(c != 0)
        def _():
            s_ref[...] = s_ref[...] + sv
            ss_ref[...] = ss_ref[...] + sq

    return _pcall(
        body,
        grid=(B, NC),
        in_specs=[
            pl.BlockSpec((1, N, C), lambda bb, cc: (bb, 0, 0)),
            pl.BlockSpec((1, CH, 1), lambda bb, cc: (bb, cc, 0)),
            pl.BlockSpec((1, CH, 3), lambda bb, cc: (bb, cc, 0)),
            pl.BlockSpec((C, Cout), lambda bb, cc: (0, 0)),
            pl.BlockSpec((1, Cout), lambda bb, cc: (0, 0)),
        ],
        out_specs=[
            pl.BlockSpec((1, CH, Cout), lambda bb, cc: (bb, cc, 0)),
            pl.BlockSpec((1, 1, Cout), lambda bb, cc: (bb, 0, 0)),
            pl.BlockSpec((1, 1, Cout), lambda bb, cc: (bb, 0, 0)),
        ],
        out_shape=[
            jax.ShapeDtypeStruct((B, R, Cout), _f32),
            jax.ShapeDtypeStruct((B, 1, Cout), _f32),
            jax.ShapeDtypeStruct((B, 1, Cout), _f32),
        ],
        compiler_params=pltpu.CompilerParams(
            dimension_semantics=("parallel", "arbitrary")),
    )(pts, knn_flat, cent_flat, W, b)


def _bn_mm(y, mean, rstd, gamma, beta, W, b, CH):
    B, R, Cin = y.shape
    Cout = W.shape[1]
    NC = R // CH

    def body(y_ref, m_ref, r_ref, g_ref, e_ref, w_ref, b_ref,
             o_ref, s_ref, ss_ref):
        c = pl.program_id(1)
        x = y_ref[0]
        h = (x - m_ref[...]) * r_ref[...] * g_ref[...] + e_ref[...]
        h = jnp.maximum(h, 0.0)
        y2 = jnp.dot(h, w_ref[...], preferred_element_type=_f32) + b_ref[...]
        o_ref[...] = y2[None]
        sv = jnp.sum(y2, axis=0, keepdims=True)[None]
        sq = jnp.sum(y2 * y2, axis=0, keepdims=True)[None]

        @pl.when(c == 0)
        def _():
            s_ref[...] = sv
            ss_ref[...] = sq

        @pl.when(c != 0)
        def _():
            s_ref[...] = s_ref[...] + sv
            ss_ref[...] = ss_ref[...] + sq

    return _pcall(
        body,
        grid=(B, NC),
        in_specs=[
            pl.BlockSpec((1, CH, Cin), lambda bb, cc: (bb, cc, 0)),
            pl.BlockSpec((1, Cin), lambda bb, cc: (0, 0)),
            pl.BlockSpec((1, Cin), lambda bb, cc: (0, 0)),
            pl.BlockSpec((1, Cin), lambda bb, cc: (0, 0)),
            pl.BlockSpec((1, Cin), lambda bb, cc: (0, 0)),
            pl.BlockSpec((Cin, Cout), lambda bb, cc: (0, 0)),
            pl.BlockSpec((1, Cout), lambda bb, cc: (0, 0)),
        ],
        out_specs=[
            pl.BlockSpec((1, CH, Cout), lambda bb, cc: (bb, cc, 0)),
            pl.BlockSpec((1, 1, Cout), lambda bb, cc: (bb, 0, 0)),
            pl.BlockSpec((1, 1, Cout), lambda bb, cc: (bb, 0, 0)),
        ],
        out_shape=[
            jax.ShapeDtypeStruct((B, R, Cout), _f32),
            jax.ShapeDtypeStruct((B, 1, Cout), _f32),
            jax.ShapeDtypeStruct((B, 1, Cout), _f32),
        ],
        compiler_params=pltpu.CompilerParams(
            dimension_semantics=("parallel", "arbitrary")),
    )(y, mean, rstd, gamma, beta, W, b)


def _bn_pool(y, mean, rstd, gamma, beta, S, K):
    B, R, C = y.shape

    def body(y_ref, m_ref, r_ref, g_ref, e_ref, o_ref):
        x = y_ref[0]
        h = (x - m_ref[...]) * r_ref[...] * g_ref[...] + e_ref[...]
        h = jnp.maximum(h, 0.0)
        o_ref[...] = jnp.max(h.reshape(S, K, C), axis=1)[None]

    return _pcall(
        body,
        grid=(B,),
        in_specs=[
            pl.BlockSpec((1, R, C), lambda bb: (bb, 0, 0)),
            pl.BlockSpec((1, C), lambda bb: (0, 0)),
            pl.BlockSpec((1, C), lambda bb: (0, 0)),
            pl.BlockSpec((1, C), lambda bb: (0, 0)),
            pl.BlockSpec((1, C), lambda bb: (0, 0)),
        ],
        out_specs=pl.BlockSpec((1, S, C), lambda bb: (bb, 0, 0)),
        out_shape=jax.ShapeDtypeStruct((B, S, C), _f32),
        compiler_params=pltpu.CompilerParams(
            dimension_semantics=("parallel",)),
    )(y, mean, rstd, gamma, beta)


def _tail(nx2, f2, sa3, head):
    B, S, _ = nx2.shape
    (w1, b1, g1, e1), (w2, b2, g2, e2), (w3, b3, g3, e3) = sa3
    (h1w, h1b, h1g, h1e), (h2w, h2b, h2g, h2e), (h3w, h3b, _, _) = head

    def bn_all(ymat):
        mean = jnp.mean(ymat, axis=0, keepdims=True)
        var = jnp.mean((ymat - mean) * (ymat - mean), axis=0, keepdims=True)
        return mean, lax.rsqrt(var + 1e-5)

    def body(nx_ref, f_ref,
             w1_ref, b1_ref, g1_ref, e1_ref,
             w2_ref, b2_ref, g2_ref, e2_ref,
             w3_ref, b3_ref, g3_ref, e3_ref,
             h1w_ref, h1b_ref, h1g_ref, h1e_ref,
             h2w_ref, h2b_ref, h2g_ref, h2e_ref,
             h3w_ref, h3b_ref, o_ref):
        g = jnp.concatenate([nx_ref[...], f_ref[...]], axis=2)
        x = g.reshape(B * S, g.shape[2])

        for w_r, b_r, g_r, e_r in (
                (w1_ref, b1_ref, g1_ref, e1_ref),
                (w2_ref, b2_ref, g2_ref, e2_ref),
                (w3_ref, b3_ref, g3_ref, e3_ref)):
            x = jnp.dot(x, w_r[...], preferred_element_type=_f32) + b_r[...]
            mean, rstd = bn_all(x)
            x = jnp.maximum((x - mean) * rstd * g_r[...] + e_r[...], 0.0)

        x = jnp.max(x.reshape(B, S, x.shape[1]), axis=1)

        for w_r, b_r, g_r, e_r in (
                (h1w_ref, h1b_ref, h1g_ref, h1e_ref),
                (h2w_ref, h2b_ref, h2g_ref, h2e_ref)):
            x = jnp.dot(x, w_r[...], preferred_element_type=_f32) + b_r[...]
            mean, rstd = bn_all(x)
            x = (x - mean) * rstd * g_r[...] + e_r[...]

        x = jnp.dot(x, h3w_ref[...], preferred_element_type=_f32) + h3b_ref[...]
        x = x - jnp.max(x, axis=1, keepdims=True)
        x = x - jnp.log(jnp.sum(jnp.exp(x), axis=1, keepdims=True))
        o_ref[...] = x

    args = (nx2, f2,
            w1, b1.reshape(1, -1), g1.reshape(1, -1), e1.reshape(1, -1),
            w2, b2.reshape(1, -1), g2.reshape(1, -1), e2.reshape(1, -1),
            w3, b3.reshape(1, -1), g3.reshape(1, -1), e3.reshape(1, -1),
            h1w, h1b.reshape(1, -1), h1g.reshape(1, -1), h1e.reshape(1, -1),
            h2w, h2b.reshape(1, -1), h2g.reshape(1, -1), h2e.reshape(1, -1),
            h3w, h3b.reshape(1, -1))
    return _pcall(
        body,
        out_shape=jax.ShapeDtypeStruct((B, h3w.shape[1]), _f32),
    )(*args)


def _stats(s, ss, n):
    tot = jnp.sum(s, axis=0)
    tot2 = jnp.sum(ss, axis=0)
    mean = tot / n
    var = tot2 / n - mean * mean
    return mean, lax.rsqrt(var + 1e-5)


def _sa_stage(pts_xyz, pts_feats, layers, npoint, K, CH):
    B, N, _ = pts_xyz.shape
    fps_idx = _fps(pts_xyz, npoint)
    knn, new_xyz = _knn(pts_xyz.transpose(0, 2, 1),
                        fps_idx.reshape(B, npoint, 1), npoint, K)
    R = npoint * K
    cent = jnp.broadcast_to(new_xyz[:, :, None, :],
                            (B, npoint, K, 3)).reshape(B, R, 3)

    (w1, b1, g1, e1) = layers[0]
    gidx = (knn.reshape(B, R)
            + (jnp.arange(B, dtype=_i32) * N)[:, None]).reshape(B * R)
    pfw = _feat_mm(pts_feats, w1[3:])
    Cout = pfw.shape[2]
    PAD = ((Cout + 127) // 128) * 128
    ftab = pfw if PAD == Cout else jnp.concatenate(
        [pfw, jnp.zeros((B, N, PAD - Cout), _f32)], axis=2)
    xtab = jnp.concatenate(
        [pts_xyz, jnp.zeros((B, N, 125), _f32)], axis=2)
    gf = _sc_gather(ftab.reshape(B * N, PAD), gidx, PAD)
    gx = _sc_gather(xtab.reshape(B * N, 128), gidx, 128)
    y, s, ss = _rel_mm(gf.reshape(B, R, PAD), gx.reshape(B, R, 128),
                       cent, Cout, w1[:3], b1.reshape(1, -1), CH)
    n = B * R
    for (w, b, g, e) in layers[1:]:
        mean, rstd = _stats(s, ss, n)
        prev_g, prev_e = g1, e1
        y, s, ss = _bn_mm(y, mean, rstd, prev_g.reshape(1, -1),
                          prev_e.reshape(1, -1), w, b.reshape(1, -1), CH)
        g1, e1 = g, e
    mean, rstd = _stats(s, ss, n)
    pooled = _bn_pool(y, mean, rstd, g1.reshape(1, -1), e1.reshape(1, -1),
                      npoint, K)
    return new_xyz, pooled


def kernel(xyz, normals, params):
    sa = params['sa']
    head = params['head']
    nx1, f1 = _sa_stage(xyz, normals, sa[0], npoint=512, K=32, CH=2048)
    nx2, f2 = _sa_stage(nx1, f1, sa[1], npoint=128, K=64, CH=2048)
    return _tail(nx2, f2, sa[2], head)


# speedup vs baseline: 1.3053x; 1.0012x over previous
import functools

import jax
import jax.numpy as jnp
from jax import lax
from jax.experimental import pallas as pl
from jax.experimental.pallas import tpu as pltpu
from jax.experimental.pallas import tpu_sc as plsc

_f32 = jnp.float32
_i32 = jnp.int32


def _pcall(body, **kw):
    return pl.pallas_call(body, **kw)


def _fps(xyz, npoint):
    B, N, _ = xyz.shape
    xs = xyz[:, :, 0]
    ys = xyz[:, :, 1]
    zs = xyz[:, :, 2]

    def body(x_ref, y_ref, z_ref, out_ref):
        X = x_ref[...]
        Y = y_ref[...]
        Z = z_ref[...]
        iota = lax.broadcasted_iota(_i32, (B, N), 1)
        iop = lax.broadcasted_iota(_i32, (B, npoint), 1)

        def step(i, state):
            dists, far, idxs = state
            idxs = idxs + ((iop == i).astype(_i32)
                           * jnp.broadcast_to(far, (B, npoint)))
            sel = iota == jnp.broadcast_to(far, (B, N))
            cx = jnp.sum(jnp.where(sel, X, 0.0), axis=1, keepdims=True)
            cy = jnp.sum(jnp.where(sel, Y, 0.0), axis=1, keepdims=True)
            cz = jnp.sum(jnp.where(sel, Z, 0.0), axis=1, keepdims=True)
            dx = X - cx
            dy = Y - cy
            dz = Z - cz
            d = dx * dx + dy * dy + dz * dz
            dists = jnp.minimum(dists, d)
            m = jnp.max(dists, axis=1, keepdims=True)
            far = jnp.min(jnp.where(dists == jnp.broadcast_to(m, (B, N)),
                                    iota, N), axis=1,
                          keepdims=True).astype(_i32)
            return dists, far, idxs

        dists0 = X * 0.0 + 1e10
        far0 = (jnp.max(X * 0.0, axis=1, keepdims=True)).astype(_i32)
        idxs0 = (X[:, :npoint] * 0.0).astype(_i32)
        _, _, idxs = lax.fori_loop(0, npoint, step, (dists0, far0, idxs0))
        out_ref[...] = idxs

    return _pcall(
        body,
        out_shape=jax.ShapeDtypeStruct((B, npoint), _i32),
    )(xs, ys, zs)


def _knn(xyzT, fps_col, S, K):
    B, _, N = xyzT.shape

    def body(xt_ref, fi_ref, knn_ref, nx_ref):
        A = xt_ref[0]
        X = A[0:1, :]
        Y = A[1:2, :]
        Z = A[2:3, :]
        idx = fi_ref[0]
        iotaSN = lax.broadcasted_iota(_i32, (S, N), 1)
        sel = iotaSN == jnp.broadcast_to(idx, (S, N))
        Xb = jnp.broadcast_to(X, (S, N))
        Yb = jnp.broadcast_to(Y, (S, N))
        Zb = jnp.broadcast_to(Z, (S, N))
        cx = jnp.sum(jnp.where(sel, Xb, 0.0), axis=1, keepdims=True)
        cy = jnp.sum(jnp.where(sel, Yb, 0.0), axis=1, keepdims=True)
        cz = jnp.sum(jnp.where(sel, Zb, 0.0), axis=1, keepdims=True)
        C = jnp.concatenate([cx, cy, cz], axis=1)
        dot = lax.dot_general(C, A, (((1,), (0,)), ((), ())),
                              preferred_element_type=_f32)
        sqC = jnp.sum(C * C, axis=1, keepdims=True)
        X2 = A * A
        sqX = (X2[0:1] + X2[1:2]) + X2[2:3]
        D = (-2.0 * dot + sqC) + sqX
        iotaK = lax.broadcasted_iota(_i32, (S, K), 1)

        def step(k, st):
            D, knn = st
            m = jnp.min(D, axis=1, keepdims=True)
            j = jnp.min(jnp.where(D == jnp.broadcast_to(m, (S, N)),
                                  iotaSN, N), axis=1,
                        keepdims=True).astype(_i32)
            knn = knn + ((iotaK == k).astype(_i32)
                         * jnp.broadcast_to(j, (S, K)))
            D = jnp.where(iotaSN == jnp.broadcast_to(j, (S, N)), jnp.inf, D)
            return D, knn

        knn0 = (D[:, :K] * 0.0).astype(_i32)
        _, knn = lax.fori_loop(0, K, step, (D, knn0))
        knn_ref[...] = knn[None]
        nx_ref[...] = C[None]

    return _pcall(
        body,
        grid=(B,),
        in_specs=[
            pl.BlockSpec((1, 3, N), lambda b: (b, 0, 0)),
            pl.BlockSpec((1, S, 1), lambda b: (b, 0, 0)),
        ],
        out_specs=[
            pl.BlockSpec((1, S, K), lambda b: (b, 0, 0)),
            pl.BlockSpec((1, S, 3), lambda b: (b, 0, 0)),
        ],
        out_shape=[
            jax.ShapeDtypeStruct((B, S, K), _i32),
            jax.ShapeDtypeStruct((B, S, 3), _f32),
        ],
        compiler_params=pltpu.CompilerParams(
            dimension_semantics=("parallel",)),
    )(xyzT, fps_col)


def _sc_gather(table, idx, C):
    M = idx.shape[0]
    info = plsc.get_sparse_core_info()
    NW = info.num_cores * info.num_subcores
    per_w = M // NW
    SUB = 128
    n_sub = per_w // SUB
    mesh = plsc.VectorSubcoreMesh(core_axis_name="c", subcore_axis_name="s")

    @functools.partial(
        pl.kernel, mesh=mesh,
        out_type=jax.ShapeDtypeStruct((M, C), _f32),
        scratch_types=[
            pltpu.VMEM((SUB,), _i32),
            pltpu.VMEM((SUB, C), _f32),
            pltpu.SemaphoreType.DMA,
        ],
    )
    def k(table_hbm, idx_hbm, out_hbm, idx_v, rows_v, sem):
        wid = lax.axis_index("s") * info.num_cores + lax.axis_index("c")
        base = wid * per_w

        def body(i, carry):
            off = base + i * SUB
            pltpu.sync_copy(idx_hbm.at[pl.ds(off, SUB)], idx_v)
            pltpu.async_copy(table_hbm.at[idx_v], rows_v, sem).wait()
            pltpu.sync_copy(rows_v, out_hbm.at[pl.ds(off, SUB)])
            return carry

        lax.fori_loop(0, n_sub, body, 0)

    return k(table, idx)


def _feat_mm(x, W):
    B, N, F = x.shape
    C = W.shape[1]

    def body(x_ref, w_ref, o_ref):
        o_ref[...] = jnp.dot(x_ref[0], w_ref[...],
                             preferred_element_type=_f32)[None]

    return _pcall(
        body,
        grid=(B,),
        in_specs=[pl.BlockSpec((1, N, F), lambda b: (b, 0, 0)),
                  pl.BlockSpec((F, C), lambda b: (0, 0))],
        out_specs=pl.BlockSpec((1, N, C), lambda b: (b, 0, 0)),
        out_shape=jax.ShapeDtypeStruct((B, N, C), _f32),
        compiler_params=pltpu.CompilerParams(
            dimension_semantics=("parallel",)),
    )(x, W)


def _rel_mm(gf, gx, cent_flat, Cout, W3, b, CH):
    B, R, PAD = gf.shape
    NC = R // CH

    def body(gf_ref, gx_ref, c_ref, w_ref, b_ref, y_ref, s_ref, ss_ref):
        c = pl.program_id(1)
        rel = gx_ref[0][:, :3] - c_ref[0]
        y = (jnp.dot(rel, w_ref[...], preferred_element_type=_f32)
             + gf_ref[0][:, :Cout]) + b_ref[...]
        y_ref[...] = y[None]
        sv = jnp.sum(y, axis=0, keepdims=True)[None]
        sq = jnp.sum(y * y, axis=0, keepdims=True)[None]

        @pl.when(c == 0)
        def _():
            s_ref[...] = sv
            ss_ref[...] = sq

        @pl.when(c != 0)
        def _():
            s_ref[...] = s_ref[...] + sv
            ss_ref[...] = ss_ref[...] + sq

    return _pcall(
        body,
        grid=(B, NC),
        in_specs=[
            pl.BlockSpec((1, CH, PAD), lambda bb, cc: (bb, cc, 0)),
            pl.BlockSpec((1, CH, 128), lambda bb, cc: (bb, cc, 0)),
            pl.BlockSpec((1, CH, 3), lambda bb, cc: (bb, cc, 0)),
            pl.BlockSpec((3, Cout), lambda bb, cc: (0, 0)),
            pl.BlockSpec((1, Cout), lambda bb, cc: (0, 0)),
        ],
        out_specs=[
            pl.BlockSpec((1, CH, Cout), lambda bb, cc: (bb, cc, 0)),
            pl.BlockSpec((1, 1, Cout), lambda bb, cc: (bb, 0, 0)),
            pl.BlockSpec((1, 1, Cout), lambda bb, cc: (bb, 0, 0)),
        ],
        out_shape=[
            jax.ShapeDtypeStruct((B, R, Cout), _f32),
            jax.ShapeDtypeStruct((B, 1, Cout), _f32),
            jax.ShapeDtypeStruct((B, 1, Cout), _f32),
        ],
        compiler_params=pltpu.CompilerParams(
            dimension_semantics=("parallel", "arbitrary")),
    )(gf, gx, cent_flat, W3, b)


def _bn_mm(y, mean, rstd, gamma, beta, W, b, CH):
    B, R, Cin = y.shape
    Cout = W.shape[1]
    NC = R // CH

    def body(y_ref, m_ref, r_ref, g_ref, e_ref, w_ref, b_ref,
             o_ref, s_ref, ss_ref):
        c = pl.program_id(1)
        x = y_ref[0]
        h = (x - m_ref[...]) * r_ref[...] * g_ref[...] + e_ref[...]
        h = jnp.maximum(h, 0.0)
        y2 = jnp.dot(h, w_ref[...], preferred_element_type=_f32) + b_ref[...]
        o_ref[...] = y2[None]
        sv = jnp.sum(y2, axis=0, keepdims=True)[None]
        sq = jnp.sum(y2 * y2, axis=0, keepdims=True)[None]

        @pl.when(c == 0)
        def _():
            s_ref[...] = sv
            ss_ref[...] = sq

        @pl.when(c != 0)
        def _():
            s_ref[...] = s_ref[...] + sv
            ss_ref[...] = ss_ref[...] + sq

    return _pcall(
        body,
        grid=(B, NC),
        in_specs=[
            pl.BlockSpec((1, CH, Cin), lambda bb, cc: (bb, cc, 0)),
            pl.BlockSpec((1, Cin), lambda bb, cc: (0, 0)),
            pl.BlockSpec((1, Cin), lambda bb, cc: (0, 0)),
            pl.BlockSpec((1, Cin), lambda bb, cc: (0, 0)),
            pl.BlockSpec((1, Cin), lambda bb, cc: (0, 0)),
            pl.BlockSpec((Cin, Cout), lambda bb, cc: (0, 0)),
            pl.BlockSpec((1, Cout), lambda bb, cc: (0, 0)),
        ],
        out_specs=[
            pl.BlockSpec((1, CH, Cout), lambda bb, cc: (bb, cc, 0)),
            pl.BlockSpec((1, 1, Cout), lambda bb, cc: (bb, 0, 0)),
            pl.BlockSpec((1, 1, Cout), lambda bb, cc: (bb, 0, 0)),
        ],
        out_shape=[
            jax.ShapeDtypeStruct((B, R, Cout), _f32),
            jax.ShapeDtypeStruct((B, 1, Cout), _f32),
            jax.ShapeDtypeStruct((B, 1, Cout), _f32),
        ],
        compiler_params=pltpu.CompilerParams(
            dimension_semantics=("parallel", "arbitrary")),
    )(y, mean, rstd, gamma, beta, W, b)


def _bn_pool(y, mean, rstd, gamma, beta, S, K):
    B, R, C = y.shape

    def body(y_ref, m_ref, r_ref, g_ref, e_ref, o_ref):
        x = y_ref[0]
        h = (x - m_ref[...]) * r_ref[...] * g_ref[...] + e_ref[...]
        h = jnp.maximum(h, 0.0)
        o_ref[...] = jnp.max(h.reshape(S, K, C), axis=1)[None]

    return _pcall(
        body,
        grid=(B,),
        in_specs=[
            pl.BlockSpec((1, R, C), lambda bb: (bb, 0, 0)),
            pl.BlockSpec((1, C), lambda bb: (0, 0)),
            pl.BlockSpec((1, C), lambda bb: (0, 0)),
            pl.BlockSpec((1, C), lambda bb: (0, 0)),
            pl.BlockSpec((1, C), lambda bb: (0, 0)),
        ],
        out_specs=pl.BlockSpec((1, S, C), lambda bb: (bb, 0, 0)),
        out_shape=jax.ShapeDtypeStruct((B, S, C), _f32),
        compiler_params=pltpu.CompilerParams(
            dimension_semantics=("parallel",)),
    )(y, mean, rstd, gamma, beta)


def _tail(nx2, f2, sa3, head):
    B, S, _ = nx2.shape
    (w1, b1, g1, e1), (w2, b2, g2, e2), (w3, b3, g3, e3) = sa3
    (h1w, h1b, h1g, h1e), (h2w, h2b, h2g, h2e), (h3w, h3b, _, _) = head

    def bn_all(ymat):
        mean = jnp.mean(ymat, axis=0, keepdims=True)
        var = jnp.mean((ymat - mean) * (ymat - mean), axis=0, keepdims=True)
        return mean, lax.rsqrt(var + 1e-5)

    def body(nx_ref, f_ref,
             w1_ref, b1_ref, g1_ref, e1_ref,
             w2_ref, b2_ref, g2_ref, e2_ref,
             w3_ref, b3_ref, g3_ref, e3_ref,
             h1w_ref, h1b_ref, h1g_ref, h1e_ref,
             h2w_ref, h2b_ref, h2g_ref, h2e_ref,
             h3w_ref, h3b_ref, o_ref):
        g = jnp.concatenate([nx_ref[...], f_ref[...]], axis=2)
        x = g.reshape(B * S, g.shape[2])

        for w_r, b_r, g_r, e_r in (
                (w1_ref, b1_ref, g1_ref, e1_ref),
                (w2_ref, b2_ref, g2_ref, e2_ref),
                (w3_ref, b3_ref, g3_ref, e3_ref)):
            x = jnp.dot(x, w_r[...], preferred_element_type=_f32) + b_r[...]
            mean, rstd = bn_all(x)
            x = jnp.maximum((x - mean) * rstd * g_r[...] + e_r[...], 0.0)

        x = jnp.max(x.reshape(B, S, x.shape[1]), axis=1)

        for w_r, b_r, g_r, e_r in (
                (h1w_ref, h1b_ref, h1g_ref, h1e_ref),
                (h2w_ref, h2b_ref, h2g_ref, h2e_ref)):
            x = jnp.dot(x, w_r[...], preferred_element_type=_f32) + b_r[...]
            mean, rstd = bn_all(x)
            x = (x - mean) * rstd * g_r[...] + e_r[...]

        x = jnp.dot(x, h3w_ref[...], preferred_element_type=_f32) + h3b_ref[...]
        x = x - jnp.max(x, axis=1, keepdims=True)
        x = x - jnp.log(jnp.sum(jnp.exp(x), axis=1, keepdims=True))
        o_ref[...] = x

    args = (nx2, f2,
            w1, b1.reshape(1, -1), g1.reshape(1, -1), e1.reshape(1, -1),
            w2, b2.reshape(1, -1), g2.reshape(1, -1), e2.reshape(1, -1),
            w3, b3.reshape(1, -1), g3.reshape(1, -1), e3.reshape(1, -1),
            h1w, h1b.reshape(1, -1), h1g.reshape(1, -1), h1e.reshape(1, -1),
            h2w, h2b.reshape(1, -1), h2g.reshape(1, -1), h2e.reshape(1, -1),
            h3w, h3b.reshape(1, -1))
    return _pcall(
        body,
        out_shape=jax.ShapeDtypeStruct((B, h3w.shape[1]), _f32),
    )(*args)


def _stats(s, ss, n):
    tot = jnp.sum(s, axis=0)
    tot2 = jnp.sum(ss, axis=0)
    mean = tot / n
    var = tot2 / n - mean * mean
    return mean, lax.rsqrt(var + 1e-5)


def _sa_stage(pts_xyz, pts_feats, layers, npoint, K, CH):
    B, N, _ = pts_xyz.shape
    fps_idx = _fps(pts_xyz, npoint)
    knn, new_xyz = _knn(pts_xyz.transpose(0, 2, 1),
                        fps_idx.reshape(B, npoint, 1), npoint, K)
    R = npoint * K
    cent = jnp.broadcast_to(new_xyz[:, :, None, :],
                            (B, npoint, K, 3)).reshape(B, R, 3)

    (w1, b1, g1, e1) = layers[0]
    gidx = (knn.reshape(B, R)
            + (jnp.arange(B, dtype=_i32) * N)[:, None]).reshape(B * R)
    pfw = _feat_mm(pts_feats, w1[3:])
    Cout = pfw.shape[2]
    PAD = ((Cout + 127) // 128) * 128
    ftab = pfw if PAD == Cout else jnp.concatenate(
        [pfw, jnp.zeros((B, N, PAD - Cout), _f32)], axis=2)
    xtab = jnp.concatenate(
        [pts_xyz, jnp.zeros((B, N, 125), _f32)], axis=2)
    gf = _sc_gather(ftab.reshape(B * N, PAD), gidx, PAD)
    gx = _sc_gather(xtab.reshape(B * N, 128), gidx, 128)
    y, s, ss = _rel_mm(gf.reshape(B, R, PAD), gx.reshape(B, R, 128),
                       cent, Cout, w1[:3], b1.reshape(1, -1), CH)
    n = B * R
    for (w, b, g, e) in layers[1:]:
        mean, rstd = _stats(s, ss, n)
        prev_g, prev_e = g1, e1
        y, s, ss = _bn_mm(y, mean, rstd, prev_g.reshape(1, -1),
                          prev_e.reshape(1, -1), w, b.reshape(1, -1), CH)
        g1, e1 = g, e
    mean, rstd = _stats(s, ss, n)
    pooled = _bn_pool(y, mean, rstd, g1.reshape(1, -1), e1.reshape(1, -1),
                      npoint, K)
    return new_xyz, pooled


def kernel(xyz, normals, params):
    sa = params['sa']
    head = params['head']
    nx1, f1 = _sa_stage(xyz, normals, sa[0], npoint=512, K=32, CH=2048)
    nx2, f2 = _sa_stage(nx1, f1, sa[1], npoint=128, K=64, CH=2048)
    return _tail(nx2, f2, sa[2], head)
